# Initial kernel scaffold; baseline (speedup 1.0000x reference)
import jax
import jax.numpy as jnp
from jax.experimental import pallas as pl


def kernel(x, freqs_cis, Wq, Wk, Wv, Wo, q_g, q_b, k_g, k_b, gate_w, gate_b):
    raise NotImplementedError("write your pallas kernel here")


# speedup vs baseline: 1.5825x; 1.5825x over previous
import math

import jax
import jax.numpy as jnp
import numpy as np
from jax.experimental import pallas as pl
from jax.experimental.pallas import tpu as pltpu

S = 2048
DIM = 768
NH = 12
HD = DIM // NH
NE = 8
HALF = HD // 2
RBLK = 512
LN_EPS = 1e-5
SCALE = 1.0 / math.sqrt(HD)


def _swap_halves(v):
    lane = jax.lax.broadcasted_iota(jnp.int32, v.shape, 1)
    left = pltpu.roll(v, DIM - HALF, axis=1)
    right = pltpu.roll(v, HALF, axis=1)
    return jnp.where(jnp.bitwise_and(lane, HD - 1) < HALF, left, right)


def _moe_attn_kernel(x_ref, cos_ref, sin_ref, gw_ref, gb_ref,
                     wq_ref, wk_ref, wv_ref, wo_ref,
                     qg_ref, qb_ref, kg_ref, kb_ref,
                     out_ref, qr, kr, vb, acc):
    e = pl.program_id(0)

    def proj_chunk(r, w_ref, g_ref, b_ref, dst, rotate):
        rows = pl.ds(r * RBLK, RBLK)
        p = jnp.dot(x_ref[rows, :], w_ref[0],
                    preferred_element_type=jnp.float32)
        if g_ref is not None:
            mu = jnp.mean(p, axis=-1, keepdims=True)
            var = jnp.mean((p - mu) ** 2, axis=-1, keepdims=True)
            p = (p - mu) * jax.lax.rsqrt(var + LN_EPS) * g_ref[0] + b_ref[0]
        if rotate:
            cos = cos_ref[rows, :].astype(jnp.float32)
            sin = sin_ref[rows, :].astype(jnp.float32)
            p = p * cos + _swap_halves(p) * sin
        dst[rows, :] = p.astype(jnp.bfloat16)

    def qkv_body(r, carry):
        proj_chunk(r, wq_ref, qg_ref, qb_ref, qr, True)
        proj_chunk(r, wk_ref, kg_ref, kb_ref, kr, True)
        proj_chunk(r, wv_ref, None, None, vb, False)
        return carry

    jax.lax.fori_loop(0, S // RBLK, qkv_body, 0)

    for h in range(NH):
        sl = slice(h * HD, (h + 1) * HD)
        kh = kr[:, sl]
        vh = vb[:, sl]

        def attn_body(qb, carry, kh=kh, vh=vh, sl=sl):
            rows = pl.ds(qb * RBLK, RBLK)
            qh = qr[rows, sl]
            s = jax.lax.dot_general(
                qh, kh, (((1,), (1,)), ((), ())),
                preferred_element_type=jnp.float32) * SCALE
            m = jnp.max(s, axis=-1, keepdims=True)
            p = jnp.exp(s - m)
            l = jnp.sum(p, axis=-1, keepdims=True)
            o = jnp.dot(p.astype(jnp.bfloat16), vh,
                        preferred_element_type=jnp.float32)
            acc[rows, sl] = o / l
            return carry

        jax.lax.fori_loop(0, S // RBLK, attn_body, 0)

    @pl.when(e == 0)
    def _():
        out_ref[...] = jnp.zeros_like(out_ref)

    def out_body(r, carry):
        rows = pl.ds(r * RBLK, RBLK)
        gall = jax.nn.sigmoid(
            jnp.dot(x_ref[rows, :], gw_ref[...],
                    preferred_element_type=jnp.float32) + gb_ref[...])
        eoh = jax.lax.broadcasted_iota(jnp.int32, (1, NE), 1) == e
        gcol = jnp.sum(jnp.where(eoh, gall, 0.0), axis=1, keepdims=True)
        o = jnp.dot(acc[rows, :].astype(jnp.bfloat16), wo_ref[0],
                    preferred_element_type=jnp.float32)
        out_ref[rows, :] += o * gcol
        return carry

    jax.lax.fori_loop(0, S // RBLK, out_body, 0)


def _build_perm():
    perm = np.zeros(DIM, dtype=np.int32)
    for h in range(NH):
        base = h * HD
        for j in range(HALF):
            perm[base + j] = base + 2 * j
            perm[base + HALF + j] = base + 2 * j + 1
    return perm


_PERM = _build_perm()


def kernel(x, freqs_cis, Wq, Wk, Wv, Wo, q_g, q_b, k_g, k_b, gate_w, gate_b):
    xb = x[0].astype(jnp.bfloat16)

    wq = Wq[:, :, _PERM].astype(jnp.bfloat16)
    wk = Wk[:, :, _PERM].astype(jnp.bfloat16)
    wv = Wv.astype(jnp.bfloat16)
    wo = Wo.astype(jnp.bfloat16)
    qg = q_g[:, _PERM].reshape(NE, 1, DIM)
    qb = q_b[:, _PERM].reshape(NE, 1, DIM)
    kg = k_g[:, _PERM].reshape(NE, 1, DIM)
    kb = k_b[:, _PERM].reshape(NE, 1, DIM)
    gw = gate_w.astype(jnp.bfloat16)
    gb = gate_b.reshape(1, NE)

    cos_ = freqs_cis[:, :, 0]
    sin_ = freqs_cis[:, :, 1]
    cosf = jnp.tile(jnp.concatenate([cos_, cos_], axis=1),
                    (1, NH)).astype(jnp.bfloat16)
    sinf = jnp.tile(jnp.concatenate([-sin_, sin_], axis=1),
                    (1, NH)).astype(jnp.bfloat16)

    full = lambda *_: (0, 0)
    per_e = lambda e: (e, 0, 0)

    out = pl.pallas_call(
        _moe_attn_kernel,
        grid=(NE,),
        in_specs=[
            pl.BlockSpec((S, DIM), full),
            pl.BlockSpec((S, DIM), full),
            pl.BlockSpec((S, DIM), full),
            pl.BlockSpec((DIM, NE), full),
            pl.BlockSpec((1, NE), full),
            pl.BlockSpec((1, DIM, DIM), per_e),
            pl.BlockSpec((1, DIM, DIM), per_e),
            pl.BlockSpec((1, DIM, DIM), per_e),
            pl.BlockSpec((1, DIM, DIM), per_e),
            pl.BlockSpec((1, 1, DIM), per_e),
            pl.BlockSpec((1, 1, DIM), per_e),
            pl.BlockSpec((1, 1, DIM), per_e),
            pl.BlockSpec((1, 1, DIM), per_e),
        ],
        out_specs=pl.BlockSpec((S, DIM), full),
        out_shape=jax.ShapeDtypeStruct((S, DIM), jnp.float32),
        scratch_shapes=[
            pltpu.VMEM((S, DIM), jnp.bfloat16),
            pltpu.VMEM((S, DIM), jnp.bfloat16),
            pltpu.VMEM((S, DIM), jnp.bfloat16),
            pltpu.VMEM((S, DIM), jnp.float32),
        ],
        compiler_params=pltpu.CompilerParams(
            dimension_semantics=("arbitrary",)),
    )(xb, cosf, sinf, gw, gb, wq, wk, wv, wo, qg, qb, kg, kb)

    return out[None]


# speedup vs baseline: 2.1748x; 1.3742x over previous
import math

import jax
import jax.numpy as jnp
import numpy as np
from jax.experimental import pallas as pl
from jax.experimental.pallas import tpu as pltpu

S = 2048
DIM = 768
NH = 12
HD = DIM // NH
HP = 128
DIMP = NH * HP
NE = 8
HALF = HD // 2
RBLK = 256
LN_EPS = 1e-5
SCALE = 1.0 / math.sqrt(HD)
INV_DIM = 1.0 / DIM


def _swap_halves(v):
    lane = jax.lax.broadcasted_iota(jnp.int32, v.shape, 1)
    left = pltpu.roll(v, v.shape[1] - HALF, axis=1)
    right = pltpu.roll(v, HALF, axis=1)
    return jnp.where(jnp.bitwise_and(lane, HD - 1) < HALF, left, right)


def _moe_attn_kernel(x_ref, cs_ref, gw_ref, gb_ref,
                     wq_ref, wk_ref, wv_ref, wo_ref,
                     qg_ref, qb_ref, kg_ref, kb_ref,
                     out_ref, qr, kr, vb, acc):
    e = pl.program_id(0)

    def ln(p, g_ref, b_ref):
        mu = jnp.sum(p, axis=-1, keepdims=True) * INV_DIM
        ex2 = jnp.sum(p * p, axis=-1, keepdims=True) * INV_DIM
        var = ex2 - mu * mu
        return (p - mu) * jax.lax.rsqrt(var + LN_EPS) * g_ref[0] + b_ref[0]

    @pl.when(e == 0)
    def _():
        lane = jax.lax.broadcasted_iota(jnp.int32, (S, DIMP), 1)
        vb[...] = jnp.where(jnp.bitwise_and(lane, HP - 1) == HD,
                            1.0, 0.0).astype(jnp.bfloat16)

    def qkv_body(r, carry):
        rows = pl.ds(r * RBLK, RBLK)
        xc = x_ref[rows, :]
        cs = cs_ref[rows, :].astype(jnp.float32)
        sw = _swap_halves(cs)
        lane = jax.lax.broadcasted_iota(jnp.int32, cs.shape, 1)
        first = jnp.bitwise_and(lane, HD - 1) < HALF
        cos = jnp.where(first, cs, sw)
        sin = jnp.where(first, -sw, cs)

        q = ln(jnp.dot(xc, wq_ref[0], preferred_element_type=jnp.float32),
               qg_ref, qb_ref)
        qr[rows, :] = ((q * cos + _swap_halves(q) * sin)
                       * SCALE).astype(jnp.bfloat16)
        k = ln(jnp.dot(xc, wk_ref[0], preferred_element_type=jnp.float32),
               kg_ref, kb_ref)
        kc = (k * cos + _swap_halves(k) * sin).astype(jnp.bfloat16)
        v = jnp.dot(xc, wv_ref[0],
                    preferred_element_type=jnp.float32).astype(jnp.bfloat16)
        for h in range(NH):
            kr[rows, h * HP:h * HP + HD] = kc[:, h * HD:(h + 1) * HD]
            vb[rows, h * HP:h * HP + HD] = v[:, h * HD:(h + 1) * HD]
        return carry

    jax.lax.fori_loop(0, S // RBLK, qkv_body, 0)

    for h in range(NH):
        kh = kr[:, h * HP:h * HP + HD]
        vh = vb[:, h * HP:(h + 1) * HP]

        def attn_blk(rows, kh=kh, vh=vh, h=h):
            qh = qr[rows, h * HD:(h + 1) * HD]
            s = jax.lax.dot_general(
                qh, kh, (((1,), (1,)), ((), ())),
                preferred_element_type=jnp.float32)
            m = jnp.max(s, axis=-1, keepdims=True)
            p = jnp.exp(s - m).astype(jnp.bfloat16)
            o = jnp.dot(p, vh, preferred_element_type=jnp.float32)
            l = o[:, HD:HD + 1]
            acc[rows, h * HD:(h + 1) * HD] = (o[:, :HD] / l).astype(jnp.bfloat16)

        def attn_body(i, carry, blk=attn_blk):
            blk(pl.ds(i * 2 * RBLK, RBLK))
            blk(pl.ds((i * 2 + 1) * RBLK, RBLK))
            return carry

        jax.lax.fori_loop(0, S // (2 * RBLK), attn_body, 0)

    @pl.when(e == 0)
    def _():
        out_ref[...] = jnp.zeros_like(out_ref)

    def out_body(r, carry):
        rows = pl.ds(r * RBLK, RBLK)
        gall = jax.nn.sigmoid(
            jnp.dot(x_ref[rows, :], gw_ref[...],
                    preferred_element_type=jnp.float32) + gb_ref[...])
        eoh = jax.lax.broadcasted_iota(jnp.int32, (1, NE), 1) == e
        gcol = jnp.sum(jnp.where(eoh, gall, 0.0), axis=1, keepdims=True)
        o = jnp.dot(acc[rows, :], wo_ref[0],
                    preferred_element_type=jnp.float32)
        out_ref[rows, :] += o * gcol
        return carry

    jax.lax.fori_loop(0, S // RBLK, out_body, 0)


def _build_perm():
    perm = np.zeros(DIM, dtype=np.int32)
    for h in range(NH):
        base = h * HD
        for j in range(HALF):
            perm[base + j] = base + 2 * j
            perm[base + HALF + j] = base + 2 * j + 1
    return perm


_PERM = _build_perm()


def kernel(x, freqs_cis, Wq, Wk, Wv, Wo, q_g, q_b, k_g, k_b, gate_w, gate_b):
    xb = x[0].astype(jnp.bfloat16)

    wq = Wq[:, :, _PERM].astype(jnp.bfloat16)
    wk = Wk[:, :, _PERM].astype(jnp.bfloat16)
    wv = Wv.astype(jnp.bfloat16)
    wo = Wo.astype(jnp.bfloat16)
    qg = q_g[:, _PERM].reshape(NE, 1, DIM)
    qb = q_b[:, _PERM].reshape(NE, 1, DIM)
    kg = k_g[:, _PERM].reshape(NE, 1, DIM)
    kb = k_b[:, _PERM].reshape(NE, 1, DIM)
    gw = gate_w.astype(jnp.bfloat16)
    gb = gate_b.reshape(1, NE)

    cos_ = freqs_cis[:, :, 0]
    sin_ = freqs_cis[:, :, 1]
    csf = jnp.tile(jnp.concatenate([cos_, sin_], axis=1),
                   (1, NH)).astype(jnp.bfloat16)

    full = lambda *_: (0, 0)
    per_e = lambda e: (e, 0, 0)

    out = pl.pallas_call(
        _moe_attn_kernel,
        grid=(NE,),
        in_specs=[
            pl.BlockSpec((S, DIM), full),
            pl.BlockSpec((S, DIM), full),
            pl.BlockSpec((DIM, NE), full),
            pl.BlockSpec((1, NE), full),
            pl.BlockSpec((1, DIM, DIM), per_e),
            pl.BlockSpec((1, DIM, DIM), per_e),
            pl.BlockSpec((1, DIM, DIM), per_e),
            pl.BlockSpec((1, DIM, DIM), per_e),
            pl.BlockSpec((1, 1, DIM), per_e),
            pl.BlockSpec((1, 1, DIM), per_e),
            pl.BlockSpec((1, 1, DIM), per_e),
            pl.BlockSpec((1, 1, DIM), per_e),
        ],
        out_specs=pl.BlockSpec((S, DIM), full),
        out_shape=jax.ShapeDtypeStruct((S, DIM), jnp.float32),
        scratch_shapes=[
            pltpu.VMEM((S, DIM), jnp.bfloat16),
            pltpu.VMEM((S, DIMP), jnp.bfloat16),
            pltpu.VMEM((S, DIMP), jnp.bfloat16),
            pltpu.VMEM((S, DIM), jnp.bfloat16),
        ],
        compiler_params=pltpu.CompilerParams(
            dimension_semantics=("arbitrary",)),
    )(xb, csf, gw, gb, wq, wk, wv, wo, qg, qb, kg, kb)

    return out[None]


# speedup vs baseline: 2.2301x; 1.0254x over previous
import math

import jax
import jax.numpy as jnp
import numpy as np
from jax.experimental import pallas as pl
from jax.experimental.pallas import tpu as pltpu

S = 2048
DIM = 768
NH = 12
HD = DIM // NH
HP = 128
DIMP = NH * HP
NE = 8
HALF = HD // 2
RBLK = 256
LN_EPS = 1e-5
QSCALE = math.log2(math.e) / math.sqrt(HD)
INV_DIM = 1.0 / DIM


def _swap_halves(v):
    lane = jax.lax.broadcasted_iota(jnp.int32, v.shape, 1)
    left = pltpu.roll(v, v.shape[1] - HALF, axis=1)
    right = pltpu.roll(v, HALF, axis=1)
    return jnp.where(jnp.bitwise_and(lane, HD - 1) < HALF, left, right)


def _moe_attn_kernel(x_ref, cs_ref, gw_ref, gb_ref,
                     wq_ref, wk_ref, wv_ref, wo_ref,
                     qg_ref, qb_ref, kg_ref, kb_ref,
                     out_ref, qr, kr, vb, acc, cosb, sinb):
    e = pl.program_id(0)

    def ln(p, g_ref, b_ref):
        mu = jnp.sum(p, axis=-1, keepdims=True) * INV_DIM
        ex2 = jnp.sum(p * p, axis=-1, keepdims=True) * INV_DIM
        var = ex2 - mu * mu
        return (p - mu) * jax.lax.rsqrt(var + LN_EPS) * g_ref[0] + b_ref[0]

    @pl.when(e == 0)
    def _():
        lane = jax.lax.broadcasted_iota(jnp.int32, (S, DIMP), 1)
        vb[...] = jnp.where(jnp.bitwise_and(lane, HP - 1) == HD,
                            1.0, 0.0).astype(jnp.bfloat16)
        cs = cs_ref[...]
        sw = _swap_halves(cs)
        lane = jax.lax.broadcasted_iota(jnp.int32, cs.shape, 1)
        first = jnp.bitwise_and(lane, HD - 1) < HALF
        cosb[...] = jnp.where(first, cs, sw)
        sinb[...] = jnp.where(first, -sw, cs)

    def qkv_body(r, carry):
        rows = pl.ds(r * RBLK, RBLK)
        xc = x_ref[rows, :]
        cos = cosb[rows, :].astype(jnp.float32)
        sin = sinb[rows, :].astype(jnp.float32)

        q = ln(jnp.dot(xc, wq_ref[0], preferred_element_type=jnp.float32),
               qg_ref, qb_ref)
        qr[rows, :] = ((q * cos + _swap_halves(q) * sin)
                       * QSCALE).astype(jnp.bfloat16)
        k = ln(jnp.dot(xc, wk_ref[0], preferred_element_type=jnp.float32),
               kg_ref, kb_ref)
        kr[rows, :] = (k * cos + _swap_halves(k) * sin).astype(jnp.bfloat16)
        v = jnp.dot(xc, wv_ref[0],
                    preferred_element_type=jnp.float32).astype(jnp.bfloat16)
        for h in range(NH):
            vb[rows, h * HP:h * HP + HD] = v[:, h * HD:(h + 1) * HD]
        return carry

    jax.lax.fori_loop(0, S // RBLK, qkv_body, 0)

    for h in range(NH):
        kh = kr[:, h * HD:(h + 1) * HD]
        vh = vb[:, h * HP:(h + 1) * HP]

        def attn_blk(rows, kh=kh, vh=vh, h=h):
            qh = qr[rows, h * HD:(h + 1) * HD]
            s = jax.lax.dot_general(
                qh, kh, (((1,), (1,)), ((), ())),
                preferred_element_type=jnp.float32)
            m = jnp.max(s, axis=-1, keepdims=True)
            p = jnp.exp2(s - m).astype(jnp.bfloat16)
            o = jnp.dot(p, vh, preferred_element_type=jnp.float32)
            l = o[:, HD:HD + 1]
            acc[rows, h * HD:(h + 1) * HD] = (o[:, :HD] / l).astype(jnp.bfloat16)

        def attn_body(i, carry, blk=attn_blk):
            blk(pl.ds(i * 2 * RBLK, RBLK))
            blk(pl.ds((i * 2 + 1) * RBLK, RBLK))
            return carry

        jax.lax.fori_loop(0, S // (2 * RBLK), attn_body, 0)

    @pl.when(e == 0)
    def _():
        out_ref[...] = jnp.zeros_like(out_ref)

    def out_body(r, carry):
        rows = pl.ds(r * RBLK, RBLK)
        gall = jax.nn.sigmoid(
            jnp.dot(x_ref[rows, :], gw_ref[...],
                    preferred_element_type=jnp.float32) + gb_ref[...])
        eoh = jax.lax.broadcasted_iota(jnp.int32, (1, NE), 1) == e
        gcol = jnp.sum(jnp.where(eoh, gall, 0.0), axis=1, keepdims=True)
        o = jnp.dot(acc[rows, :], wo_ref[0],
                    preferred_element_type=jnp.float32)
        out_ref[rows, :] += o * gcol
        return carry

    jax.lax.fori_loop(0, S // RBLK, out_body, 0)


def _build_perm():
    perm = np.zeros(DIM, dtype=np.int32)
    for h in range(NH):
        base = h * HD
        for j in range(HALF):
            perm[base + j] = base + 2 * j
            perm[base + HALF + j] = base + 2 * j + 1
    return perm


_PERM = _build_perm()


def kernel(x, freqs_cis, Wq, Wk, Wv, Wo, q_g, q_b, k_g, k_b, gate_w, gate_b):
    xb = x[0].astype(jnp.bfloat16)

    wq = Wq[:, :, _PERM].astype(jnp.bfloat16)
    wk = Wk[:, :, _PERM].astype(jnp.bfloat16)
    wv = Wv.astype(jnp.bfloat16)
    wo = Wo.astype(jnp.bfloat16)
    qg = q_g[:, _PERM].reshape(NE, 1, DIM)
    qb = q_b[:, _PERM].reshape(NE, 1, DIM)
    kg = k_g[:, _PERM].reshape(NE, 1, DIM)
    kb = k_b[:, _PERM].reshape(NE, 1, DIM)
    gw = gate_w.astype(jnp.bfloat16)
    gb = gate_b.reshape(1, NE)

    cos_ = freqs_cis[:, :, 0]
    sin_ = freqs_cis[:, :, 1]
    csf = jnp.tile(jnp.concatenate([cos_, sin_], axis=1),
                   (1, NH)).astype(jnp.bfloat16)

    full = lambda *_: (0, 0)
    per_e = lambda e: (e, 0, 0)

    out = pl.pallas_call(
        _moe_attn_kernel,
        grid=(NE,),
        in_specs=[
            pl.BlockSpec((S, DIM), full),
            pl.BlockSpec((S, DIM), full),
            pl.BlockSpec((DIM, NE), full),
            pl.BlockSpec((1, NE), full),
            pl.BlockSpec((1, DIM, DIM), per_e),
            pl.BlockSpec((1, DIM, DIM), per_e),
            pl.BlockSpec((1, DIM, DIM), per_e),
            pl.BlockSpec((1, DIM, DIM), per_e),
            pl.BlockSpec((1, 1, DIM), per_e),
            pl.BlockSpec((1, 1, DIM), per_e),
            pl.BlockSpec((1, 1, DIM), per_e),
            pl.BlockSpec((1, 1, DIM), per_e),
        ],
        out_specs=pl.BlockSpec((S, DIM), full),
        out_shape=jax.ShapeDtypeStruct((S, DIM), jnp.float32),
        scratch_shapes=[
            pltpu.VMEM((S, DIM), jnp.bfloat16),
            pltpu.VMEM((S, DIM), jnp.bfloat16),
            pltpu.VMEM((S, DIMP), jnp.bfloat16),
            pltpu.VMEM((S, DIM), jnp.bfloat16),
            pltpu.VMEM((S, DIM), jnp.bfloat16),
            pltpu.VMEM((S, DIM), jnp.bfloat16),
        ],
        compiler_params=pltpu.CompilerParams(
            dimension_semantics=("arbitrary",)),
    )(xb, csf, gw, gb, wq, wk, wv, wo, qg, qb, kg, kb)

    return out[None]
